# baseline (speedup 1.0000x reference)
import functools

import jax
import jax.numpy as jnp
from jax import lax
from jax.experimental import pallas as pl
from jax.experimental.pallas import tpu as pltpu
from jax.experimental.pallas import tpu_sc as plsc

_NC, _NS = 2, 16
_NW = _NC * _NS
_C = 64
_LANES = 16


def kernel(x, pe):
    batch, seq_len, d_model = x.shape
    rows = batch * seq_len
    xf = x.reshape(rows * d_model)
    pef = pe.reshape(-1)
    rows_per_w = rows // _NW
    nchunk = rows_per_w // _C
    chunk_words = _C * d_model
    mesh = plsc.VectorSubcoreMesh(core_axis_name="c", subcore_axis_name="s")

    @functools.partial(
        pl.kernel,
        mesh=mesh,
        out_type=jax.ShapeDtypeStruct((rows * d_model,), jnp.float32),
        scratch_types=[
            pltpu.VMEM((chunk_words,), jnp.float32),
            pltpu.VMEM((chunk_words,), jnp.float32),
        ],
    )
    def k(x_hbm, pe_hbm, out_hbm, xbuf, pebuf):
        cid = lax.axis_index("c")
        sid = lax.axis_index("s")
        wid = sid * _NC + cid
        base = wid * rows_per_w
        for i in range(nchunk):
            row0 = base + i * _C
            off = row0 * d_model
            s_off = lax.rem(row0, seq_len) * d_model
            pltpu.sync_copy(x_hbm.at[pl.ds(off, chunk_words)], xbuf)
            pltpu.sync_copy(pe_hbm.at[pl.ds(s_off, chunk_words)], pebuf)

            @pl.loop(0, chunk_words // _LANES, unroll=8)
            def _add(j):
                sl = pl.ds(j * _LANES, _LANES)
                xbuf[sl] = xbuf[sl] + pebuf[sl]

            pltpu.sync_copy(xbuf, out_hbm.at[pl.ds(off, chunk_words)])

    out = k(xf, pef)
    return out.reshape(batch, seq_len, d_model)


# speedup vs baseline: 8.3456x; 8.3456x over previous
import jax
import jax.numpy as jnp
from jax.experimental import pallas as pl

_S_BLK = 512


def _add_pe_body(x_ref, pe_ref, o_ref):
    o_ref[...] = x_ref[...] + pe_ref[...][None, :, :]


def kernel(x, pe):
    batch, seq_len, d_model = x.shape
    pe = pe[:seq_len]
    grid = (seq_len // _S_BLK,)
    return pl.pallas_call(
        _add_pe_body,
        grid=grid,
        in_specs=[
            pl.BlockSpec((batch, _S_BLK, d_model), lambda i: (0, i, 0)),
            pl.BlockSpec((_S_BLK, d_model), lambda i: (i, 0)),
        ],
        out_specs=pl.BlockSpec((batch, _S_BLK, d_model), lambda i: (0, i, 0)),
        out_shape=jax.ShapeDtypeStruct(x.shape, x.dtype),
    )(x, pe)
